# baseline (speedup 1.0000x reference)
import jax
import jax.numpy as jnp
import numpy as np
from jax.experimental import pallas as pl


def _widen_f32_to_f64_words(val):
    bits = jax.lax.bitcast_convert_type(val, jnp.int32)
    sign = bits & np.int32(-0x80000000)
    mag = bits & np.int32(0x7FFFFFFF)
    hi = sign | ((mag >> 3) + np.int32(0x38000000))
    hi = jnp.where(mag >= np.int32(0x7F800000), sign | np.int32(0x7FF80000), hi)
    lo = bits << 29
    return lo, hi


def _pairwise_entropy_kernel(x1_ref, x2_ref, lo_ref, hi_ref):
    x1 = x1_ref[...]
    x2 = x2_ref[...]
    a = jnp.concatenate([x1 * jnp.log(x1), x1], axis=1)
    b = jnp.concatenate([x2, x2 * jnp.log(x2)], axis=1)
    d = -jax.lax.dot_general(
        a, b, (((1,), (1,)), ((), ())), preferred_element_type=jnp.float32
    )
    lo, hi = _widen_f32_to_f64_words(d)
    lo_ref[...] = lo
    hi_ref[...] = hi


def kernel(x1, x2):
    n1 = x1.shape[2]
    n2 = x2.shape[2]
    k = x1.shape[3]
    x1f = x1.reshape(n1, k)
    x2f = x2.reshape(n2, k)
    lo, hi = pl.pallas_call(
        _pairwise_entropy_kernel,
        out_shape=(
            jax.ShapeDtypeStruct((n1, n2), jnp.int32),
            jax.ShapeDtypeStruct((n1, n2), jnp.int32),
        ),
    )(x1f, x2f)
    words = jnp.stack([lo, hi], axis=-1)
    return jax.lax.bitcast_convert_type(words, jnp.float64)


# speedup vs baseline: 1.0860x; 1.0860x over previous
import jax
import jax.numpy as jnp
import numpy as np
from jax.experimental import pallas as pl


def _widen_f32_to_f64_words(val):
    bits = jax.lax.bitcast_convert_type(val, jnp.int32)
    sign = bits & np.int32(-0x80000000)
    mag = bits & np.int32(0x7FFFFFFF)
    hi = sign | ((mag >> 3) + np.int32(0x38000000))
    hi = jnp.where(mag >= np.int32(0x7F800000), sign | np.int32(0x7FF80000), hi)
    lo = bits << 29
    return lo, hi


def _pairwise_entropy_kernel(x1_ref, x2_ref, lo_ref, hi_ref):
    x1 = x1_ref[...]
    x2 = x2_ref[...]
    a = jnp.concatenate([x1 * jnp.log(x1), x1], axis=1)
    b = jnp.concatenate([x2, x2 * jnp.log(x2)], axis=1)
    d = -jax.lax.dot_general(
        a, b, (((1,), (1,)), ((), ())), preferred_element_type=jnp.float32
    )
    lo, hi = _widen_f32_to_f64_words(d)
    lo_ref[...] = lo
    hi_ref[...] = hi


def kernel(x1, x2):
    n1 = x1.shape[2]
    n2 = x2.shape[2]
    k = x1.shape[3]
    x1f = x1.reshape(n1, k)
    x2f = x2.reshape(n2, k)
    lo, hi = pl.pallas_call(
        _pairwise_entropy_kernel,
        out_shape=(
            jax.ShapeDtypeStruct((n1, n2), jnp.int32),
            jax.ShapeDtypeStruct((n1, n2), jnp.int32),
        ),
    )(x1f, x2f)
    lo64 = jax.lax.bitcast_convert_type(lo, jnp.uint32).astype(jnp.uint64)
    hi64 = jax.lax.bitcast_convert_type(hi, jnp.uint32).astype(jnp.uint64)
    return jax.lax.bitcast_convert_type((hi64 << 32) | lo64, jnp.float64)


# speedup vs baseline: 1.4518x; 1.3368x over previous
import jax
import jax.numpy as jnp
from jax.experimental import pallas as pl


def _pairwise_entropy_kernel(x1_ref, x2_ref, o_ref):
    x1 = x1_ref[...]
    x2 = x2_ref[...]
    a = jnp.concatenate([x1 * jnp.log(x1), x1], axis=1)
    b = jnp.concatenate([x2, x2 * jnp.log(x2)], axis=1)
    o_ref[...] = -jax.lax.dot_general(
        a, b, (((1,), (1,)), ((), ())), preferred_element_type=jnp.float32
    )


def kernel(x1, x2):
    n1 = x1.shape[2]
    n2 = x2.shape[2]
    k = x1.shape[3]
    x1f = x1.reshape(n1, k)
    x2f = x2.reshape(n2, k)
    out = pl.pallas_call(
        _pairwise_entropy_kernel,
        out_shape=jax.ShapeDtypeStruct((n1, n2), jnp.float32),
    )(x1f, x2f)
    return out.astype(jnp.float64)
